# Initial kernel scaffold; baseline (speedup 1.0000x reference)
import jax
import jax.numpy as jnp
from jax.experimental import pallas as pl


def kernel(input_ids, emb_table, W):
    raise NotImplementedError("write your pallas kernel here")


# speedup vs baseline: 1.1470x; 1.1470x over previous
import functools

import jax
import jax.numpy as jnp
from jax import lax
from jax.experimental import pallas as pl
from jax.experimental.pallas import tpu as pltpu
from jax.experimental.pallas import tpu_sc as plsc

NUM_BUCKETS = 1000000
EMBED_DIM = 64
PROJ_DIM = 128
BATCH = 1024
SEQ = 200
TOK = BATCH * SEQ

_BN = 4096
_NBLK = -(-NUM_BUCKETS // _BN)
_PROWS = _NBLK * _BN


def _proj_body(t_ref, w_ref, p_ref):
    p_ref[...] = lax.dot_general(
        t_ref[...], w_ref[...], (((0,), (1,)), ((), ())),
        preferred_element_type=jnp.float32)


_project_table = pl.pallas_call(
    _proj_body,
    grid=(_NBLK,),
    in_specs=[
        pl.BlockSpec((EMBED_DIM, _BN), lambda i: (jnp.int32(0), i)),
        pl.BlockSpec((PROJ_DIM, EMBED_DIM),
                     lambda i: (jnp.int32(0), jnp.int32(0))),
    ],
    out_specs=pl.BlockSpec((_BN, PROJ_DIM), lambda i: (i, jnp.int32(0))),
    out_shape=jax.ShapeDtypeStruct((_PROWS, PROJ_DIM), jnp.float32),
)

_info = plsc.get_sparse_core_info()
_NC, _NS, _L = _info.num_cores, _info.num_subcores, _info.num_lanes
NW = _NC * _NS
TPW = TOK // NW
GCHUNK = 128
KFIRE = 2
ROWS_PER_BUF = KFIRE * GCHUNK
NSUPER = TPW // ROWS_PER_BUF

_mesh = plsc.VectorSubcoreMesh(core_axis_name="c", subcore_axis_name="s")


@functools.partial(
    pl.kernel,
    mesh=_mesh,
    out_type=jax.ShapeDtypeStruct((TOK, PROJ_DIM), jnp.float32),
    scratch_types=[
        pltpu.VMEM((TPW,), jnp.int32),
        pltpu.VMEM((TPW,), jnp.int32),
        pltpu.VMEM((TPW,), jnp.int32),
        pltpu.VMEM((ROWS_PER_BUF, PROJ_DIM), jnp.float32),
        pltpu.VMEM((ROWS_PER_BUF, PROJ_DIM), jnp.float32),
        pltpu.SemaphoreType.DMA,
        pltpu.SemaphoreType.DMA,
        pltpu.SemaphoreType.DMA,
        pltpu.SemaphoreType.DMA,
    ],
)
def _sc_hash_gather(cur_hbm, prev_hbm, p_hbm, out_hbm,
                    cur_v, prev_v, idx_v, buf0, buf1,
                    sem_g0, sem_g1, sem_w0, sem_w1):
    wid = lax.axis_index("s") * jnp.int32(_NC) + lax.axis_index("c")
    base = pl.multiple_of(wid * jnp.int32(TPW), 8)
    pltpu.sync_copy(cur_hbm.at[pl.ds(base, TPW)], cur_v)
    pltpu.sync_copy(prev_hbm.at[pl.ds(base, TPW)], prev_v)

    def hash_step(i, carry):
        off = pl.multiple_of(i * jnp.int32(_L), 8)
        cur = cur_v[pl.ds(off, _L)]
        prev = prev_v[pl.ds(off, _L)]
        cmod = jnp.int32(1000000)
        hi = lax.shift_right_logical(prev, jnp.int32(7))
        lo = lax.bitwise_and(prev, jnp.int32(127))
        t = lax.rem(hi * jnp.int32(777088) + lo * jnp.int32(131071), cmod)
        idx_v[pl.ds(off, _L)] = lax.rem(t + cur, cmod)
        return carry

    lax.fori_loop(jnp.int32(0), jnp.int32(TPW // _L), hash_step, jnp.int32(0))

    def do_super(s, buf, sem_g, sem_w):
        row0 = pl.multiple_of(s * jnp.int32(ROWS_PER_BUF), 8)
        cps = []
        for k in range(KFIRE):
            off = pl.multiple_of(row0 + jnp.int32(k * GCHUNK), 8)
            cps.append(pltpu.async_copy(
                p_hbm.at[idx_v.at[pl.ds(off, GCHUNK)]],
                buf.at[pl.ds(k * GCHUNK, GCHUNK)],
                sem_g))
        for cp in cps:
            cp.wait()
        return pltpu.async_copy(
            buf, out_hbm.at[pl.ds(base + row0, ROWS_PER_BUF)], sem_w)

    def super_pair(i, carry):
        s0 = i * jnp.int32(2)
        wb0 = do_super(s0, buf0, sem_g0, sem_w0)
        wb1 = do_super(s0 + jnp.int32(1), buf1, sem_g1, sem_w1)
        wb0.wait()
        wb1.wait()
        return carry

    lax.fori_loop(jnp.int32(0), jnp.int32(NSUPER // 2), super_pair, jnp.int32(0))
    wb = do_super(jnp.int32(NSUPER - 1), buf0, sem_g0, sem_w0)
    wb.wait()


def kernel(input_ids, emb_table, W):
    ids32 = input_ids.astype(jnp.int32)
    cur = ids32.reshape(-1)
    prev = jnp.pad(ids32[:, :-1], ((0, 0), (1, 0))).reshape(-1)
    proj = _project_table(emb_table.T, W)
    out = _sc_hash_gather(cur, prev, proj)
    return out.reshape(BATCH, SEQ, PROJ_DIM)


# speedup vs baseline: 1.1825x; 1.0310x over previous
import functools

import jax
import jax.numpy as jnp
from jax import lax
from jax.experimental import pallas as pl
from jax.experimental.pallas import tpu as pltpu
from jax.experimental.pallas import tpu_sc as plsc

NUM_BUCKETS = 1000000
EMBED_DIM = 64
PROJ_DIM = 128
BATCH = 1024
SEQ = 200
TOK = BATCH * SEQ

_BN = 4096
_NBLK = -(-NUM_BUCKETS // _BN)
_PROWS = _NBLK * _BN


def _proj_body(t_ref, w_ref, p_ref):
    p_ref[...] = lax.dot_general(
        t_ref[...].astype(jnp.bfloat16), w_ref[...].astype(jnp.bfloat16),
        (((0,), (1,)), ((), ())),
        preferred_element_type=jnp.float32)


_project_table = pl.pallas_call(
    _proj_body,
    grid=(_NBLK,),
    in_specs=[
        pl.BlockSpec((EMBED_DIM, _BN), lambda i: (jnp.int32(0), i)),
        pl.BlockSpec((PROJ_DIM, EMBED_DIM),
                     lambda i: (jnp.int32(0), jnp.int32(0))),
    ],
    out_specs=pl.BlockSpec((_BN, PROJ_DIM), lambda i: (i, jnp.int32(0))),
    out_shape=jax.ShapeDtypeStruct((_PROWS, PROJ_DIM), jnp.float32),
)

_info = plsc.get_sparse_core_info()
_NC, _NS, _L = _info.num_cores, _info.num_subcores, _info.num_lanes
NW = _NC * _NS
TPW = TOK // NW
GCHUNK = 128
KFIRE = 2
ROWS_PER_BUF = KFIRE * GCHUNK
NSUPER = TPW // ROWS_PER_BUF

_mesh = plsc.VectorSubcoreMesh(core_axis_name="c", subcore_axis_name="s")


@functools.partial(
    pl.kernel,
    mesh=_mesh,
    out_type=jax.ShapeDtypeStruct((TOK, PROJ_DIM), jnp.float32),
    scratch_types=[
        pltpu.VMEM((TPW,), jnp.int32),
        pltpu.VMEM((TPW,), jnp.int32),
        pltpu.VMEM((TPW,), jnp.int32),
        pltpu.VMEM((ROWS_PER_BUF, PROJ_DIM), jnp.float32),
        pltpu.VMEM((ROWS_PER_BUF, PROJ_DIM), jnp.float32),
        pltpu.SemaphoreType.DMA,
        pltpu.SemaphoreType.DMA,
        pltpu.SemaphoreType.DMA,
        pltpu.SemaphoreType.DMA,
    ],
)
def _sc_hash_gather(cur_hbm, prev_hbm, p_hbm, out_hbm,
                    cur_v, prev_v, idx_v, buf0, buf1,
                    sem_g0, sem_g1, sem_w0, sem_w1):
    wid = lax.axis_index("s") * jnp.int32(_NC) + lax.axis_index("c")
    base = pl.multiple_of(wid * jnp.int32(TPW), 8)
    pltpu.sync_copy(cur_hbm.at[pl.ds(base, TPW)], cur_v)
    pltpu.sync_copy(prev_hbm.at[pl.ds(base, TPW)], prev_v)

    def hash_step(i, carry):
        off = pl.multiple_of(i * jnp.int32(_L), 8)
        cur = cur_v[pl.ds(off, _L)]
        prev = prev_v[pl.ds(off, _L)]
        cmod = jnp.int32(1000000)
        hi = lax.shift_right_logical(prev, jnp.int32(7))
        lo = lax.bitwise_and(prev, jnp.int32(127))
        t = lax.rem(hi * jnp.int32(777088) + lo * jnp.int32(131071), cmod)
        idx_v[pl.ds(off, _L)] = lax.rem(t + cur, cmod)
        return carry

    lax.fori_loop(jnp.int32(0), jnp.int32(TPW // _L), hash_step, jnp.int32(0))

    def do_super(s, buf, sem_g, sem_w):
        row0 = pl.multiple_of(s * jnp.int32(ROWS_PER_BUF), 8)
        cps = []
        for k in range(KFIRE):
            off = pl.multiple_of(row0 + jnp.int32(k * GCHUNK), 8)
            cps.append(pltpu.async_copy(
                p_hbm.at[idx_v.at[pl.ds(off, GCHUNK)]],
                buf.at[pl.ds(k * GCHUNK, GCHUNK)],
                sem_g))
        for cp in cps:
            cp.wait()
        return pltpu.async_copy(
            buf, out_hbm.at[pl.ds(base + row0, ROWS_PER_BUF)], sem_w)

    def super_pair(i, carry):
        s0 = i * jnp.int32(2)
        wb0 = do_super(s0, buf0, sem_g0, sem_w0)
        wb1 = do_super(s0 + jnp.int32(1), buf1, sem_g1, sem_w1)
        wb0.wait()
        wb1.wait()
        return carry

    lax.fori_loop(jnp.int32(0), jnp.int32(NSUPER // 2), super_pair, jnp.int32(0))
    wb = do_super(jnp.int32(NSUPER - 1), buf0, sem_g0, sem_w0)
    wb.wait()


def kernel(input_ids, emb_table, W):
    ids32 = input_ids.astype(jnp.int32)
    cur = ids32.reshape(-1)
    prev = jnp.pad(ids32[:, :-1], ((0, 0), (1, 0))).reshape(-1)
    proj = _project_table(emb_table.T, W)
    out = _sc_hash_gather(cur, prev, proj)
    return out.reshape(BATCH, SEQ, PROJ_DIM)


# speedup vs baseline: 1.4203x; 1.2011x over previous
import functools

import jax
import jax.numpy as jnp
from jax import lax
from jax.experimental import pallas as pl
from jax.experimental.pallas import tpu as pltpu
from jax.experimental.pallas import tpu_sc as plsc

NUM_BUCKETS = 1000000
EMBED_DIM = 64
PROJ_DIM = 128
BATCH = 1024
SEQ = 200
TOK = BATCH * SEQ

_BN = 8192
_NBLK = -(-NUM_BUCKETS // _BN)
_PROWS = _NBLK * _BN


def _proj_body(t_ref, w_ref, p_ref):
    p_ref[...] = lax.dot_general(
        t_ref[...].astype(jnp.bfloat16), w_ref[...].astype(jnp.bfloat16),
        (((0,), (1,)), ((), ())),
        preferred_element_type=jnp.float32)


_project_table = pl.pallas_call(
    _proj_body,
    grid=(_NBLK,),
    in_specs=[
        pl.BlockSpec((EMBED_DIM, _BN), lambda i: (jnp.int32(0), i)),
        pl.BlockSpec((PROJ_DIM, EMBED_DIM),
                     lambda i: (jnp.int32(0), jnp.int32(0))),
    ],
    out_specs=pl.BlockSpec((_BN, PROJ_DIM), lambda i: (i, jnp.int32(0))),
    out_shape=jax.ShapeDtypeStruct((_PROWS, PROJ_DIM), jnp.float32),
)

_info = plsc.get_sparse_core_info()
_NC, _NS, _L = _info.num_cores, _info.num_subcores, _info.num_lanes
NW = _NC * _NS
TPW = TOK // NW
GCHUNK = 128
KFIRE = 2
ROWS_PER_BUF = KFIRE * GCHUNK
NSUPER = TPW // ROWS_PER_BUF

_mesh = plsc.VectorSubcoreMesh(core_axis_name="c", subcore_axis_name="s")


@functools.partial(
    pl.kernel,
    mesh=_mesh,
    out_type=jax.ShapeDtypeStruct((TOK, PROJ_DIM), jnp.float32),
    scratch_types=[
        pltpu.VMEM((TPW,), jnp.int32),
        pltpu.VMEM((TPW,), jnp.int32),
        pltpu.VMEM((TPW,), jnp.int32),
        pltpu.VMEM((ROWS_PER_BUF, PROJ_DIM), jnp.float32),
        pltpu.VMEM((ROWS_PER_BUF, PROJ_DIM), jnp.float32),
        pltpu.SemaphoreType.DMA,
        pltpu.SemaphoreType.DMA,
        pltpu.SemaphoreType.DMA,
        pltpu.SemaphoreType.DMA,
    ],
)
def _sc_hash_gather(cur_hbm, prev_hbm, p_hbm, out_hbm,
                    cur_v, prev_v, idx_v, buf0, buf1,
                    sem_g0, sem_g1, sem_w0, sem_w1):
    wid = lax.axis_index("s") * jnp.int32(_NC) + lax.axis_index("c")
    base = pl.multiple_of(wid * jnp.int32(TPW), 8)
    pltpu.sync_copy(cur_hbm.at[pl.ds(base, TPW)], cur_v)
    pltpu.sync_copy(prev_hbm.at[pl.ds(base, TPW)], prev_v)

    def hash_step(i, carry):
        off = pl.multiple_of(i * jnp.int32(_L), 8)
        cur = cur_v[pl.ds(off, _L)]
        prev = prev_v[pl.ds(off, _L)]
        cmod = jnp.int32(1000000)
        hi = lax.shift_right_logical(prev, jnp.int32(7))
        lo = lax.bitwise_and(prev, jnp.int32(127))
        t = lax.rem(hi * jnp.int32(777088) + lo * jnp.int32(131071), cmod)
        idx_v[pl.ds(off, _L)] = lax.rem(t + cur, cmod)
        return carry

    lax.fori_loop(jnp.int32(0), jnp.int32(TPW // _L), hash_step, jnp.int32(0))

    def fire_gathers(s, buf, sem_g):
        row0 = pl.multiple_of(s * jnp.int32(ROWS_PER_BUF), 8)
        cps = []
        for k in range(KFIRE):
            off = pl.multiple_of(row0 + jnp.int32(k * GCHUNK), 8)
            cps.append(pltpu.async_copy(
                p_hbm.at[idx_v.at[pl.ds(off, GCHUNK)]],
                buf.at[pl.ds(k * GCHUNK, GCHUNK)],
                sem_g))
        return cps

    def fire_wb(s, buf, sem_w):
        row0 = pl.multiple_of(s * jnp.int32(ROWS_PER_BUF), 8)
        pltpu.async_copy(buf, out_hbm.at[pl.ds(base + row0, ROWS_PER_BUF)],
                         sem_w)

    def drain_wb(buf, sem_w):
        pltpu.make_async_copy(
            buf, out_hbm.at[pl.ds(base, ROWS_PER_BUF)], sem_w).wait()

    def super_pair(i, carry):
        s0 = i * jnp.int32(2)

        @pl.when(i > jnp.int32(0))
        def _():
            drain_wb(buf0, sem_w0)
        g0 = fire_gathers(s0, buf0, sem_g0)

        @pl.when(i > jnp.int32(0))
        def _():
            drain_wb(buf1, sem_w1)
        g1 = fire_gathers(s0 + jnp.int32(1), buf1, sem_g1)

        for cp in g0:
            cp.wait()
        fire_wb(s0, buf0, sem_w0)
        for cp in g1:
            cp.wait()
        fire_wb(s0 + jnp.int32(1), buf1, sem_w1)
        return carry

    lax.fori_loop(jnp.int32(0), jnp.int32(NSUPER // 2), super_pair, jnp.int32(0))
    drain_wb(buf0, sem_w0)
    gt = fire_gathers(jnp.int32(NSUPER - 1), buf0, sem_g0)
    drain_wb(buf1, sem_w1)
    for cp in gt:
        cp.wait()
    fire_wb(jnp.int32(NSUPER - 1), buf0, sem_w0)
    drain_wb(buf0, sem_w0)


def kernel(input_ids, emb_table, W):
    ids32 = input_ids.astype(jnp.int32)
    cur = ids32.reshape(-1)
    prev = jnp.pad(ids32[:, :-1], ((0, 0), (1, 0))).reshape(-1)
    proj = _project_table(emb_table.T, W)
    out = _sc_hash_gather(cur, prev, proj)
    return out.reshape(BATCH, SEQ, PROJ_DIM)


# speedup vs baseline: 1.5367x; 1.0820x over previous
import functools

import jax
import jax.numpy as jnp
from jax import lax
from jax.experimental import pallas as pl
from jax.experimental.pallas import tpu as pltpu
from jax.experimental.pallas import tpu_sc as plsc

NUM_BUCKETS = 1000000
EMBED_DIM = 64
PROJ_DIM = 128
BATCH = 1024
SEQ = 200
TOK = BATCH * SEQ

_BN = 8192
_NBLK = -(-NUM_BUCKETS // _BN)
_PROWS = _NBLK * _BN


def _proj_body(t_ref, w_ref, p_ref):
    p_ref[...] = lax.dot_general(
        t_ref[...].astype(jnp.bfloat16), w_ref[...].astype(jnp.bfloat16),
        (((0,), (1,)), ((), ())),
        preferred_element_type=jnp.float32)


_project_table = pl.pallas_call(
    _proj_body,
    grid=(_NBLK,),
    in_specs=[
        pl.BlockSpec((EMBED_DIM, _BN), lambda i: (jnp.int32(0), i)),
        pl.BlockSpec((PROJ_DIM, EMBED_DIM),
                     lambda i: (jnp.int32(0), jnp.int32(0))),
    ],
    out_specs=pl.BlockSpec((_BN, PROJ_DIM), lambda i: (i, jnp.int32(0))),
    out_shape=jax.ShapeDtypeStruct((_PROWS, PROJ_DIM), jnp.float32),
)

_info = plsc.get_sparse_core_info()
_NC, _NS, _L = _info.num_cores, _info.num_subcores, _info.num_lanes
NW = _NC * _NS
TPW = TOK // NW
GCHUNK = 128
KFIRE = 2
ROWS_PER_BUF = KFIRE * GCHUNK
NSUPER = TPW // ROWS_PER_BUF

_mesh = plsc.VectorSubcoreMesh(core_axis_name="c", subcore_axis_name="s")


@functools.partial(
    pl.kernel,
    mesh=_mesh,
    out_type=jax.ShapeDtypeStruct((TOK,), jnp.int32),
    scratch_types=[
        pltpu.VMEM((TPW,), jnp.int32),
        pltpu.VMEM((TPW,), jnp.int32),
        pltpu.VMEM((TPW,), jnp.int32),
    ],
)
def _sc_hash(cur_hbm, prev_hbm, idx_hbm, cur_v, prev_v, idx_v):
    wid = lax.axis_index("s") * jnp.int32(_NC) + lax.axis_index("c")
    base = pl.multiple_of(wid * jnp.int32(TPW), 8)
    pltpu.sync_copy(cur_hbm.at[pl.ds(base, TPW)], cur_v)
    pltpu.sync_copy(prev_hbm.at[pl.ds(base, TPW)], prev_v)

    def hash_step(i, carry):
        off = pl.multiple_of(i * jnp.int32(_L), 8)
        cur = cur_v[pl.ds(off, _L)]
        prev = prev_v[pl.ds(off, _L)]
        cmod = jnp.int32(1000000)
        hi = lax.shift_right_logical(prev, jnp.int32(7))
        lo = lax.bitwise_and(prev, jnp.int32(127))
        t = lax.rem(hi * jnp.int32(777088) + lo * jnp.int32(131071), cmod)
        idx_v[pl.ds(off, _L)] = lax.rem(t + cur, cmod)
        return carry

    lax.fori_loop(jnp.int32(0), jnp.int32(TPW // _L), hash_step, jnp.int32(0))
    pltpu.sync_copy(idx_v, idx_hbm.at[pl.ds(base, TPW)])


@functools.partial(
    pl.kernel,
    mesh=_mesh,
    out_type=jax.ShapeDtypeStruct((TOK, PROJ_DIM), jnp.float32),
    scratch_types=[
        pltpu.VMEM((TPW,), jnp.int32),
        pltpu.VMEM((ROWS_PER_BUF, PROJ_DIM), jnp.float32),
        pltpu.VMEM((ROWS_PER_BUF, PROJ_DIM), jnp.float32),
        pltpu.SemaphoreType.DMA,
        pltpu.SemaphoreType.DMA,
        pltpu.SemaphoreType.DMA,
        pltpu.SemaphoreType.DMA,
    ],
)
def _sc_gather(idx_hbm, p_hbm, out_hbm,
               idx_v, buf0, buf1,
               sem_g0, sem_g1, sem_w0, sem_w1):
    wid = lax.axis_index("s") * jnp.int32(_NC) + lax.axis_index("c")
    base = pl.multiple_of(wid * jnp.int32(TPW), 8)
    pltpu.sync_copy(idx_hbm.at[pl.ds(base, TPW)], idx_v)

    def fire_gathers(s, buf, sem_g):
        row0 = pl.multiple_of(s * jnp.int32(ROWS_PER_BUF), 8)
        cps = []
        for k in range(KFIRE):
            off = pl.multiple_of(row0 + jnp.int32(k * GCHUNK), 8)
            cps.append(pltpu.async_copy(
                p_hbm.at[idx_v.at[pl.ds(off, GCHUNK)]],
                buf.at[pl.ds(k * GCHUNK, GCHUNK)],
                sem_g))
        return cps

    def fire_wb(s, buf, sem_w):
        row0 = pl.multiple_of(s * jnp.int32(ROWS_PER_BUF), 8)
        pltpu.async_copy(buf, out_hbm.at[pl.ds(base + row0, ROWS_PER_BUF)],
                         sem_w)

    def drain_wb(buf, sem_w):
        pltpu.make_async_copy(
            buf, out_hbm.at[pl.ds(base, ROWS_PER_BUF)], sem_w).wait()

    def super_pair(i, carry):
        s0 = i * jnp.int32(2)

        @pl.when(i > jnp.int32(0))
        def _():
            drain_wb(buf0, sem_w0)
        g0 = fire_gathers(s0, buf0, sem_g0)

        @pl.when(i > jnp.int32(0))
        def _():
            drain_wb(buf1, sem_w1)
        g1 = fire_gathers(s0 + jnp.int32(1), buf1, sem_g1)

        for cp in g0:
            cp.wait()
        fire_wb(s0, buf0, sem_w0)
        for cp in g1:
            cp.wait()
        fire_wb(s0 + jnp.int32(1), buf1, sem_w1)
        return carry

    lax.fori_loop(jnp.int32(0), jnp.int32(NSUPER // 2), super_pair, jnp.int32(0))
    drain_wb(buf0, sem_w0)
    gt = fire_gathers(jnp.int32(NSUPER - 1), buf0, sem_g0)
    drain_wb(buf1, sem_w1)
    for cp in gt:
        cp.wait()
    fire_wb(jnp.int32(NSUPER - 1), buf0, sem_w0)
    drain_wb(buf0, sem_w0)


def kernel(input_ids, emb_table, W):
    ids32 = input_ids.astype(jnp.int32)
    cur = ids32.reshape(-1)
    prev = jnp.pad(ids32[:, :-1], ((0, 0), (1, 0))).reshape(-1)
    idx = _sc_hash(cur, prev)
    proj = _project_table(emb_table.T, W)
    out = _sc_gather(idx, proj)
    return out.reshape(BATCH, SEQ, PROJ_DIM)


# speedup vs baseline: 1.5957x; 1.0384x over previous
import functools

import jax
import jax.numpy as jnp
from jax import lax
from jax.experimental import pallas as pl
from jax.experimental.pallas import tpu as pltpu
from jax.experimental.pallas import tpu_sc as plsc

NUM_BUCKETS = 1000000
EMBED_DIM = 64
PROJ_DIM = 128
BATCH = 1024
SEQ = 200
TOK = BATCH * SEQ

_BN = 16384
_NBLK = -(-NUM_BUCKETS // _BN)
_PROWS = _NBLK * _BN


def _proj_body(t_ref, w_ref, p_ref):
    p_ref[...] = lax.dot_general(
        t_ref[...].astype(jnp.bfloat16), w_ref[...].astype(jnp.bfloat16),
        (((0,), (1,)), ((), ())),
        preferred_element_type=jnp.float32)


_project_table = pl.pallas_call(
    _proj_body,
    grid=(_NBLK,),
    in_specs=[
        pl.BlockSpec((EMBED_DIM, _BN), lambda i: (jnp.int32(0), i)),
        pl.BlockSpec((PROJ_DIM, EMBED_DIM),
                     lambda i: (jnp.int32(0), jnp.int32(0))),
    ],
    out_specs=pl.BlockSpec((_BN, PROJ_DIM), lambda i: (i, jnp.int32(0))),
    out_shape=jax.ShapeDtypeStruct((_PROWS, PROJ_DIM), jnp.float32),
)

_info = plsc.get_sparse_core_info()
_NC, _NS, _L = _info.num_cores, _info.num_subcores, _info.num_lanes
NW = _NC * _NS
TPW = TOK // NW
GCHUNK = 128
KFIRE = 2
ROWS_PER_BUF = KFIRE * GCHUNK
NSUPER = TPW // ROWS_PER_BUF

_mesh = plsc.VectorSubcoreMesh(core_axis_name="c", subcore_axis_name="s")


@functools.partial(
    pl.kernel,
    mesh=_mesh,
    out_type=jax.ShapeDtypeStruct((TOK,), jnp.int32),
    scratch_types=[
        pltpu.VMEM((TPW,), jnp.int32),
        pltpu.VMEM((TPW,), jnp.int32),
        pltpu.VMEM((TPW,), jnp.int32),
    ],
)
def _sc_hash(cur_hbm, prev_hbm, idx_hbm, cur_v, prev_v, idx_v):
    wid = lax.axis_index("s") * jnp.int32(_NC) + lax.axis_index("c")
    base = pl.multiple_of(wid * jnp.int32(TPW), 8)
    pltpu.sync_copy(cur_hbm.at[pl.ds(base, TPW)], cur_v)
    pltpu.sync_copy(prev_hbm.at[pl.ds(base, TPW)], prev_v)

    def hash_step(i, carry):
        off = pl.multiple_of(i * jnp.int32(_L), 8)
        cur = cur_v[pl.ds(off, _L)]
        prev = prev_v[pl.ds(off, _L)]
        cmod = jnp.int32(1000000)
        hi = lax.shift_right_logical(prev, jnp.int32(7))
        lo = lax.bitwise_and(prev, jnp.int32(127))
        t = lax.rem(hi * jnp.int32(777088) + lo * jnp.int32(131071), cmod)
        idx_v[pl.ds(off, _L)] = lax.rem(t + cur, cmod)
        return carry

    lax.fori_loop(jnp.int32(0), jnp.int32(TPW // _L), hash_step, jnp.int32(0))
    pltpu.sync_copy(idx_v, idx_hbm.at[pl.ds(base, TPW)])


@functools.partial(
    pl.kernel,
    mesh=_mesh,
    out_type=jax.ShapeDtypeStruct((TOK, PROJ_DIM), jnp.float32),
    scratch_types=[
        pltpu.VMEM((TPW,), jnp.int32),
        pltpu.VMEM((ROWS_PER_BUF, PROJ_DIM), jnp.float32),
        pltpu.VMEM((ROWS_PER_BUF, PROJ_DIM), jnp.float32),
        pltpu.SemaphoreType.DMA,
        pltpu.SemaphoreType.DMA,
        pltpu.SemaphoreType.DMA,
        pltpu.SemaphoreType.DMA,
    ],
)
def _sc_gather(idx_hbm, p_hbm, out_hbm,
               idx_v, buf0, buf1,
               sem_g0, sem_g1, sem_w0, sem_w1):
    wid = lax.axis_index("s") * jnp.int32(_NC) + lax.axis_index("c")
    base = pl.multiple_of(wid * jnp.int32(TPW), 8)
    pltpu.sync_copy(idx_hbm.at[pl.ds(base, TPW)], idx_v)

    def fire_gathers(s, buf, sem_g):
        row0 = pl.multiple_of(s * jnp.int32(ROWS_PER_BUF), 8)
        cps = []
        for k in range(KFIRE):
            off = pl.multiple_of(row0 + jnp.int32(k * GCHUNK), 8)
            cps.append(pltpu.async_copy(
                p_hbm.at[idx_v.at[pl.ds(off, GCHUNK)]],
                buf.at[pl.ds(k * GCHUNK, GCHUNK)],
                sem_g))
        return cps

    def fire_wb(s, buf, sem_w):
        row0 = pl.multiple_of(s * jnp.int32(ROWS_PER_BUF), 8)
        pltpu.async_copy(buf, out_hbm.at[pl.ds(base + row0, ROWS_PER_BUF)],
                         sem_w)

    def drain_wb(buf, sem_w):
        pltpu.make_async_copy(
            buf, out_hbm.at[pl.ds(base, ROWS_PER_BUF)], sem_w).wait()

    def super_pair(i, carry):
        s0 = i * jnp.int32(2)

        @pl.when(i > jnp.int32(0))
        def _():
            drain_wb(buf0, sem_w0)
        g0 = fire_gathers(s0, buf0, sem_g0)

        @pl.when(i > jnp.int32(0))
        def _():
            drain_wb(buf1, sem_w1)
        g1 = fire_gathers(s0 + jnp.int32(1), buf1, sem_g1)

        for cp in g0:
            cp.wait()
        fire_wb(s0, buf0, sem_w0)
        for cp in g1:
            cp.wait()
        fire_wb(s0 + jnp.int32(1), buf1, sem_w1)
        return carry

    lax.fori_loop(jnp.int32(0), jnp.int32(NSUPER // 2), super_pair, jnp.int32(0))
    drain_wb(buf0, sem_w0)
    gt = fire_gathers(jnp.int32(NSUPER - 1), buf0, sem_g0)
    drain_wb(buf1, sem_w1)
    for cp in gt:
        cp.wait()
    fire_wb(jnp.int32(NSUPER - 1), buf0, sem_w0)
    drain_wb(buf0, sem_w0)


def kernel(input_ids, emb_table, W):
    ids32 = input_ids.astype(jnp.int32)
    cur = ids32.reshape(-1)
    prev = jnp.pad(ids32[:, :-1], ((0, 0), (1, 0))).reshape(-1)
    idx = _sc_hash(cur, prev)
    proj = _project_table(emb_table.T, W)
    out = _sc_gather(idx, proj)
    return out.reshape(BATCH, SEQ, PROJ_DIM)


# speedup vs baseline: 1.6385x; 1.0268x over previous
import functools

import jax
import jax.numpy as jnp
from jax import lax
from jax.experimental import pallas as pl
from jax.experimental.pallas import tpu as pltpu
from jax.experimental.pallas import tpu_sc as plsc

NUM_BUCKETS = 1000000
EMBED_DIM = 64
PROJ_DIM = 128
BATCH = 1024
SEQ = 200
TOK = BATCH * SEQ

_BN = 32768
_NBLK = -(-NUM_BUCKETS // _BN)
_PROWS = _NBLK * _BN


def _proj_body(t_ref, w_ref, p_ref):
    p_ref[...] = lax.dot_general(
        t_ref[...].astype(jnp.bfloat16), w_ref[...].astype(jnp.bfloat16),
        (((0,), (1,)), ((), ())),
        preferred_element_type=jnp.float32)


_project_table = pl.pallas_call(
    _proj_body,
    grid=(_NBLK,),
    in_specs=[
        pl.BlockSpec((EMBED_DIM, _BN), lambda i: (jnp.int32(0), i)),
        pl.BlockSpec((PROJ_DIM, EMBED_DIM),
                     lambda i: (jnp.int32(0), jnp.int32(0))),
    ],
    out_specs=pl.BlockSpec((_BN, PROJ_DIM), lambda i: (i, jnp.int32(0))),
    out_shape=jax.ShapeDtypeStruct((_PROWS, PROJ_DIM), jnp.float32),
)

_info = plsc.get_sparse_core_info()
_NC, _NS, _L = _info.num_cores, _info.num_subcores, _info.num_lanes
NW = _NC * _NS
TPW = TOK // NW
GCHUNK = 128
NBUF = 4
NG = TPW // GCHUNK
NG_MAIN = (NG // NBUF) * NBUF
NG_TAIL = NG - NG_MAIN

_mesh = plsc.VectorSubcoreMesh(core_axis_name="c", subcore_axis_name="s")


@functools.partial(
    pl.kernel,
    mesh=_mesh,
    out_type=jax.ShapeDtypeStruct((TOK,), jnp.int32),
    scratch_types=[
        pltpu.VMEM((TPW,), jnp.int32),
        pltpu.VMEM((TPW,), jnp.int32),
        pltpu.VMEM((TPW,), jnp.int32),
    ],
)
def _sc_hash(cur_hbm, prev_hbm, idx_hbm, cur_v, prev_v, idx_v):
    wid = lax.axis_index("s") * jnp.int32(_NC) + lax.axis_index("c")
    base = pl.multiple_of(wid * jnp.int32(TPW), 8)
    pltpu.sync_copy(cur_hbm.at[pl.ds(base, TPW)], cur_v)
    pltpu.sync_copy(prev_hbm.at[pl.ds(base, TPW)], prev_v)

    def hash_step(i, carry):
        off = pl.multiple_of(i * jnp.int32(_L), 8)
        cur = cur_v[pl.ds(off, _L)]
        prev = prev_v[pl.ds(off, _L)]
        cmod = jnp.int32(1000000)
        hi = lax.shift_right_logical(prev, jnp.int32(7))
        lo = lax.bitwise_and(prev, jnp.int32(127))
        t = lax.rem(hi * jnp.int32(777088) + lo * jnp.int32(131071), cmod)
        idx_v[pl.ds(off, _L)] = lax.rem(t + cur, cmod)
        return carry

    lax.fori_loop(jnp.int32(0), jnp.int32(TPW // _L), hash_step, jnp.int32(0))
    pltpu.sync_copy(idx_v, idx_hbm.at[pl.ds(base, TPW)])


@functools.partial(
    pl.kernel,
    mesh=_mesh,
    out_type=jax.ShapeDtypeStruct((TOK, PROJ_DIM), jnp.float32),
    scratch_types=(
        [pltpu.VMEM((TPW,), jnp.int32)]
        + [pltpu.VMEM((GCHUNK, PROJ_DIM), jnp.float32)] * NBUF
        + [pltpu.SemaphoreType.DMA] * (2 * NBUF)
    ),
)
def _sc_gather(idx_hbm, p_hbm, out_hbm, idx_v, *bufs_and_sems):
    bufs = bufs_and_sems[:NBUF]
    sem_g = bufs_and_sems[NBUF:2 * NBUF]
    sem_w = bufs_and_sems[2 * NBUF:]
    wid = lax.axis_index("s") * jnp.int32(_NC) + lax.axis_index("c")
    base = pl.multiple_of(wid * jnp.int32(TPW), 8)
    pltpu.sync_copy(idx_hbm.at[pl.ds(base, TPW)], idx_v)

    def fire_gather(g, j):
        off = pl.multiple_of(g * jnp.int32(GCHUNK), 8)
        return pltpu.async_copy(
            p_hbm.at[idx_v.at[pl.ds(off, GCHUNK)]], bufs[j], sem_g[j])

    def fire_wb(g, j):
        off = pl.multiple_of(g * jnp.int32(GCHUNK), 8)
        pltpu.async_copy(bufs[j], out_hbm.at[pl.ds(base + off, GCHUNK)],
                         sem_w[j])

    def drain_wb(j):
        pltpu.make_async_copy(
            bufs[j], out_hbm.at[pl.ds(base, GCHUNK)], sem_w[j]).wait()

    def ring_body(i, carry):
        g0 = i * jnp.int32(NBUF)
        cps = []
        for j in range(NBUF):
            @pl.when(i > jnp.int32(0))
            def _(j=j):
                drain_wb(j)
            cps.append(fire_gather(g0 + jnp.int32(j), j))
        for j in range(NBUF):
            cps[j].wait()
            fire_wb(g0 + jnp.int32(j), j)
        return carry

    lax.fori_loop(jnp.int32(0), jnp.int32(NG_MAIN // NBUF), ring_body,
                  jnp.int32(0))
    tail_cps = []
    for j in range(NG_TAIL):
        drain_wb(j)
        tail_cps.append(fire_gather(jnp.int32(NG_MAIN + j), j))
    for j in range(NG_TAIL, NBUF):
        drain_wb(j)
    for j in range(NG_TAIL):
        tail_cps[j].wait()
        fire_wb(jnp.int32(NG_MAIN + j), j)
    for j in range(NG_TAIL):
        drain_wb(j)


def kernel(input_ids, emb_table, W):
    ids32 = input_ids.astype(jnp.int32)
    cur = ids32.reshape(-1)
    prev = jnp.pad(ids32[:, :-1], ((0, 0), (1, 0))).reshape(-1)
    idx = _sc_hash(cur, prev)
    proj = _project_table(emb_table.T, W)
    out = _sc_gather(idx, proj)
    return out.reshape(BATCH, SEQ, PROJ_DIM)


# speedup vs baseline: 1.6457x; 1.0044x over previous
import functools

import jax
import jax.numpy as jnp
from jax import lax
from jax.experimental import pallas as pl
from jax.experimental.pallas import tpu as pltpu
from jax.experimental.pallas import tpu_sc as plsc

NUM_BUCKETS = 1000000
EMBED_DIM = 64
PROJ_DIM = 128
BATCH = 1024
SEQ = 200
TOK = BATCH * SEQ

_BN = 32768
_NBLK = -(-NUM_BUCKETS // _BN)
_PROWS = _NBLK * _BN


def _proj_body(t_ref, w_ref, p_ref):
    p_ref[...] = lax.dot_general(
        t_ref[...].astype(jnp.bfloat16), w_ref[...].astype(jnp.bfloat16),
        (((0,), (1,)), ((), ())),
        preferred_element_type=jnp.float32)


_project_table = pl.pallas_call(
    _proj_body,
    grid=(_NBLK,),
    in_specs=[
        pl.BlockSpec((EMBED_DIM, _BN), lambda i: (jnp.int32(0), i)),
        pl.BlockSpec((PROJ_DIM, EMBED_DIM),
                     lambda i: (jnp.int32(0), jnp.int32(0))),
    ],
    out_specs=pl.BlockSpec((_BN, PROJ_DIM), lambda i: (i, jnp.int32(0))),
    out_shape=jax.ShapeDtypeStruct((_PROWS, PROJ_DIM), jnp.float32),
)

_info = plsc.get_sparse_core_info()
_NC, _NS, _L = _info.num_cores, _info.num_subcores, _info.num_lanes
NW = _NC * _NS
TPW = TOK // NW
GCHUNK = 128
NBUF = 6
NG = TPW // GCHUNK
NG_MAIN = (NG // NBUF) * NBUF
NG_TAIL = NG - NG_MAIN

_mesh = plsc.VectorSubcoreMesh(core_axis_name="c", subcore_axis_name="s")


@functools.partial(
    pl.kernel,
    mesh=_mesh,
    out_type=jax.ShapeDtypeStruct((TOK,), jnp.int32),
    scratch_types=[
        pltpu.VMEM((TPW,), jnp.int32),
        pltpu.VMEM((TPW,), jnp.int32),
        pltpu.VMEM((TPW,), jnp.int32),
    ],
)
def _sc_hash(cur_hbm, prev_hbm, idx_hbm, cur_v, prev_v, idx_v):
    wid = lax.axis_index("s") * jnp.int32(_NC) + lax.axis_index("c")
    base = pl.multiple_of(wid * jnp.int32(TPW), 8)
    pltpu.sync_copy(cur_hbm.at[pl.ds(base, TPW)], cur_v)
    pltpu.sync_copy(prev_hbm.at[pl.ds(base, TPW)], prev_v)

    def hash_step(i, carry):
        off = pl.multiple_of(i * jnp.int32(_L), 8)
        cur = cur_v[pl.ds(off, _L)]
        prev = prev_v[pl.ds(off, _L)]
        cmod = jnp.int32(1000000)
        hi = lax.shift_right_logical(prev, jnp.int32(7))
        lo = lax.bitwise_and(prev, jnp.int32(127))
        t = lax.rem(hi * jnp.int32(777088) + lo * jnp.int32(131071), cmod)
        idx_v[pl.ds(off, _L)] = lax.rem(t + cur, cmod)
        return carry

    lax.fori_loop(jnp.int32(0), jnp.int32(TPW // _L), hash_step, jnp.int32(0))
    pltpu.sync_copy(idx_v, idx_hbm.at[pl.ds(base, TPW)])


@functools.partial(
    pl.kernel,
    mesh=_mesh,
    out_type=jax.ShapeDtypeStruct((TOK, PROJ_DIM), jnp.float32),
    scratch_types=(
        [pltpu.VMEM((TPW,), jnp.int32)]
        + [pltpu.VMEM((GCHUNK, PROJ_DIM), jnp.float32)] * NBUF
        + [pltpu.SemaphoreType.DMA] * (2 * NBUF)
    ),
)
def _sc_gather(idx_hbm, p_hbm, out_hbm, idx_v, *bufs_and_sems):
    bufs = bufs_and_sems[:NBUF]
    sem_g = bufs_and_sems[NBUF:2 * NBUF]
    sem_w = bufs_and_sems[2 * NBUF:]
    wid = lax.axis_index("s") * jnp.int32(_NC) + lax.axis_index("c")
    base = pl.multiple_of(wid * jnp.int32(TPW), 8)
    pltpu.sync_copy(idx_hbm.at[pl.ds(base, TPW)], idx_v)

    def fire_gather(g, j):
        off = pl.multiple_of(g * jnp.int32(GCHUNK), 8)
        return pltpu.async_copy(
            p_hbm.at[idx_v.at[pl.ds(off, GCHUNK)]], bufs[j], sem_g[j])

    def fire_wb(g, j):
        off = pl.multiple_of(g * jnp.int32(GCHUNK), 8)
        pltpu.async_copy(bufs[j], out_hbm.at[pl.ds(base + off, GCHUNK)],
                         sem_w[j])

    def drain_wb(j):
        pltpu.make_async_copy(
            bufs[j], out_hbm.at[pl.ds(base, GCHUNK)], sem_w[j]).wait()

    def ring_body(i, carry):
        g0 = i * jnp.int32(NBUF)
        cps = []
        for j in range(NBUF):
            @pl.when(i > jnp.int32(0))
            def _(j=j):
                drain_wb(j)
            cps.append(fire_gather(g0 + jnp.int32(j), j))
        for j in range(NBUF):
            cps[j].wait()
            fire_wb(g0 + jnp.int32(j), j)
        return carry

    lax.fori_loop(jnp.int32(0), jnp.int32(NG_MAIN // NBUF), ring_body,
                  jnp.int32(0))
    tail_cps = []
    for j in range(NG_TAIL):
        drain_wb(j)
        tail_cps.append(fire_gather(jnp.int32(NG_MAIN + j), j))
    for j in range(NG_TAIL, NBUF):
        drain_wb(j)
    for j in range(NG_TAIL):
        tail_cps[j].wait()
        fire_wb(jnp.int32(NG_MAIN + j), j)
    for j in range(NG_TAIL):
        drain_wb(j)


def kernel(input_ids, emb_table, W):
    ids32 = input_ids.astype(jnp.int32)
    cur = ids32.reshape(-1)
    prev = jnp.pad(ids32[:, :-1], ((0, 0), (1, 0))).reshape(-1)
    idx = _sc_hash(cur, prev)
    proj = _project_table(emb_table.T, W)
    out = _sc_gather(idx, proj)
    return out.reshape(BATCH, SEQ, PROJ_DIM)
